# baseline (speedup 1.0000x reference)
import functools

import numpy as np
import jax
import jax.numpy as jnp
from jax import lax
from jax.experimental import pallas as pl
from jax.experimental.pallas import tpu as pltpu
from jax.experimental.pallas import tpu_sc as plsc

_VOCAB = 1000000
_D = 64
_B = 1024
_L = 200

_NW = 32
_ROWS = _B * _L
_RPW = _ROWS // _NW
_CHUNK = 200
_GSUB = 100
_NCHUNK = _RPW // _CHUNK
_NG = 13


def _sinusoidal_pe_np(length, d_model):
    pos = np.arange(length, dtype=np.float32)[:, None]
    div = np.exp(
        np.arange(0, d_model, 2, dtype=np.float32) * (-np.log(10000.0) / d_model)
    )
    pe = np.zeros((length, d_model), dtype=np.float32)
    pe[:, 0::2] = np.sin(pos * div)
    pe[:, 1::2] = np.cos(pos * div)
    return pe


_mesh = plsc.VectorSubcoreMesh(core_axis_name="c", subcore_axis_name="s")


@functools.partial(
    pl.kernel,
    mesh=_mesh,
    compiler_params=pltpu.CompilerParams(
        use_tc_tiling_on_sc=False, needs_layout_passes=False),
    out_type=jax.ShapeDtypeStruct((_B, _ROWS // _B // 2, 2 * _D), jnp.float32),
    scratch_types=[
        pltpu.VMEM((2 * _NCHUNK, _GSUB), jnp.int32),
        pltpu.VMEM((_RPW + 16,), jnp.int32),
        pltpu.VMEM((_D, _L + 8), jnp.float32),
        pltpu.VMEM((2, _CHUNK, 2 * _D), jnp.float32),
        pltpu.VMEM((2, _CHUNK // 2, 2 * _D), jnp.float32),
        pltpu.SemaphoreType.DMA,
        pltpu.SemaphoreType.DMA,
        pltpu.SemaphoreType.DMA,
        pltpu.SemaphoreType.DMA,
    ],
)
def _embed_kernel(iv_hbm, off_hbm, pet_hbm, table_hbm, out_hbm,
                  iv_v, off_v, pet_v, inb, outb, g0, g1, s0, s1):
    wid = lax.axis_index("s") * 2 + lax.axis_index("c")
    bbase = wid * (_RPW // _L)
    gsem = (g0, g1)
    ssem = (s0, s1)

    pltpu.sync_copy(iv_hbm.at[wid], iv_v)
    pltpu.sync_copy(off_hbm.at[wid], off_v.at[pl.ds(0, _RPW)])
    pltpu.sync_copy(pet_hbm, pet_v)

    def gather(i, b):
        pltpu.async_copy(table_hbm.at[iv_v.at[2 * i]],
                         inb.at[b, pl.ds(0, _GSUB)], gsem[b])
        pltpu.async_copy(table_hbm.at[iv_v.at[2 * i + 1]],
                         inb.at[b, pl.ds(_GSUB, _GSUB)], gsem[b])

    def wait_gather(b):
        for _ in range(2):
            pltpu.make_async_copy(table_hbm.at[iv_v.at[0]],
                                  inb.at[b, pl.ds(0, _GSUB)], gsem[b]).wait()

    def store(i, b):
        pltpu.async_copy(outb.at[b], out_hbm.at[bbase + i], ssem[b])

    def wait_store(b):
        pltpu.make_async_copy(outb.at[b], out_hbm.at[bbase], ssem[b]).wait()

    iota = lax.iota(jnp.int32, 16)
    halfiota = lax.shift_right_logical(iota, 1)
    colbase = (iota & 1) * _D

    def select_add_pe(i, b):
        def group(g, nval):
            rows = g * 16 + iota
            mask = (iota < nval) if nval < 16 else None
            if mask is not None:
                rows = jnp.minimum(rows, _CHUNK - 1)
            offv = off_v[pl.ds(i * _CHUNK + g * 16, 16)]
            row2 = g * 8 + halfiota
            for d in range(_D):
                col = offv + d
                val = plsc.load_gather(inb.at[b], [rows, col], mask=mask)
                pev = pet_v[d, pl.ds(g * 16, 16)]
                plsc.store_scatter(outb.at[b], [row2, colbase + d], val + pev,
                                   mask=mask)

        def body(g, _):
            group(g, 16)
            return 0

        lax.fori_loop(0, _NG - 1, body, 0)
        group(_NG - 1, _CHUNK - (_NG - 1) * 16)

    gather(0, 0)
    gather(1, 1)
    for b in (0, 1):
        wait_gather(b)
        select_add_pe(b, b)
        store(b, b)
        gather(b + 2, b)

    def body(i2, _):
        for b in (0, 1):
            i = 2 * i2 + b
            wait_gather(b)
            wait_store(b)
            select_add_pe(i, b)
            store(i, b)
            gather(i + 2, b)
        return 0

    lax.fori_loop(1, _NCHUNK // 2 - 1, body, 0)

    for b in (0, 1):
        i = _NCHUNK - 2 + b
        wait_gather(b)
        wait_store(b)
        select_add_pe(i, b)
        store(i, b)
    wait_store(0)
    wait_store(1)


def kernel(sequence, token_table):
    seq = sequence.reshape(_NW, _RPW).astype(jnp.int32)
    iv = lax.shift_right_logical(seq, 1).reshape(_NW, 2 * _NCHUNK, _GSUB)
    off = (seq & 1) * _D
    pet_np = np.zeros((_D, _L + 8), dtype=np.float32)
    pet_np[:, :_L] = _sinusoidal_pe_np(_L, _D).T
    pet = jnp.asarray(pet_np)
    table2 = token_table.reshape(_VOCAB // 2, 2 * _D)
    out = _embed_kernel(iv, off, pet, table2)
    return out.reshape(_B, _L, _D)


# speedup vs baseline: 1.5927x; 1.5927x over previous
import functools

import numpy as np
import jax
import jax.numpy as jnp
from jax import lax
from jax.experimental import pallas as pl
from jax.experimental.pallas import tpu as pltpu
from jax.experimental.pallas import tpu_sc as plsc

_VOCAB = 1000000
_D = 64
_B = 1024
_L = 200

_NW = 32
_ROWS = _B * _L
_RPW = _ROWS // _NW
_CHUNK = 200
_GSUB = 100
_NCHUNK = _RPW // _CHUNK


def _sinusoidal_pe_np(length, d_model):
    pos = np.arange(length, dtype=np.float32)[:, None]
    div = np.exp(
        np.arange(0, d_model, 2, dtype=np.float32) * (-np.log(10000.0) / d_model)
    )
    pe = np.zeros((length, d_model), dtype=np.float32)
    pe[:, 0::2] = np.sin(pos * div)
    pe[:, 1::2] = np.cos(pos * div)
    return pe


_mesh = plsc.VectorSubcoreMesh(core_axis_name="c", subcore_axis_name="s")


@functools.partial(
    pl.kernel,
    mesh=_mesh,
    compiler_params=pltpu.CompilerParams(
        use_tc_tiling_on_sc=False, needs_layout_passes=False),
    out_type=jax.ShapeDtypeStruct((_B, _L // 2, 2 * _D), jnp.float32),
    scratch_types=[
        pltpu.VMEM((2 * _NCHUNK, _GSUB), jnp.int32),
        pltpu.VMEM((_L, _D), jnp.float32),
        pltpu.VMEM((2, _CHUNK, 2 * _D), jnp.float32),
        pltpu.VMEM((2, _CHUNK // 2, 2 * _D), jnp.float32),
        pltpu.SemaphoreType.DMA,
        pltpu.SemaphoreType.DMA,
        pltpu.SemaphoreType.DMA,
        pltpu.SemaphoreType.DMA,
    ],
)
def _embed_kernel(idx_hbm, pe_hbm, table_hbm, out_hbm,
                  idx_v, pe_v, inb, outb, g0, g1, s0, s1):
    wid = lax.axis_index("s") * 2 + lax.axis_index("c")
    bbase = wid * (_RPW // _L)
    gsem = (g0, g1)
    ssem = (s0, s1)

    pltpu.sync_copy(idx_hbm.at[wid], idx_v)
    pltpu.sync_copy(pe_hbm, pe_v)

    def gather(i, b):
        pltpu.async_copy(table_hbm.at[idx_v.at[2 * i]],
                         inb.at[b, pl.ds(0, _GSUB)], gsem[b])
        pltpu.async_copy(table_hbm.at[idx_v.at[2 * i + 1]],
                         inb.at[b, pl.ds(_GSUB, _GSUB)], gsem[b])

    def wait_gather(b):
        for _ in range(2):
            pltpu.make_async_copy(table_hbm.at[idx_v.at[0]],
                                  inb.at[b, pl.ds(0, _GSUB)], gsem[b]).wait()

    def store(i, b):
        pltpu.async_copy(outb.at[b], out_hbm.at[bbase + i], ssem[b])

    def wait_store(b):
        pltpu.make_async_copy(outb.at[b], out_hbm.at[bbase], ssem[b]).wait()

    def add_pe(b):
        def rowpair(m, _):
            for h in range(2):
                r = 2 * m + h
                for q in range(_D // 16):
                    sl = pl.ds(q * 16, 16)
                    osl = pl.ds(h * _D + q * 16, 16)
                    outb[b, m, osl] = inb[b, r, sl] + pe_v[r, sl]
            return 0

        lax.fori_loop(0, _CHUNK // 2, rowpair, 0)

    gather(0, 0)
    gather(1, 1)
    for b in (0, 1):
        wait_gather(b)
        add_pe(b)
        store(b, b)
        gather(b + 2, b)

    def body(i2, _):
        for b in (0, 1):
            i = 2 * i2 + b
            wait_gather(b)
            wait_store(b)
            add_pe(b)
            store(i, b)
            gather(i + 2, b)
        return 0

    lax.fori_loop(1, _NCHUNK // 2 - 1, body, 0)

    for b in (0, 1):
        i = _NCHUNK - 2 + b
        wait_gather(b)
        wait_store(b)
        add_pe(b)
        store(i, b)
    wait_store(0)
    wait_store(1)


def kernel(sequence, token_table):
    seq = sequence.reshape(-1).astype(jnp.int32).reshape(_NW, 2 * _NCHUNK, _GSUB)
    pe = jnp.asarray(_sinusoidal_pe_np(_L, _D))
    tpad = jnp.pad(token_table, ((0, 0), (0, _D)))
    out = _embed_kernel(seq, pe, tpad)
    return out.reshape(_B, _L, _D)


# speedup vs baseline: 1.7104x; 1.0739x over previous
import functools

import numpy as np
import jax
import jax.numpy as jnp
from jax import lax
from jax.experimental import pallas as pl
from jax.experimental.pallas import tpu as pltpu
from jax.experimental.pallas import tpu_sc as plsc

_VOCAB = 1000000
_D = 64
_B = 1024
_L = 200

_NW = 32
_ROWS = _B * _L
_RPW = _ROWS // _NW
_CHUNK = 400
_GSUB = 100
_NCHUNK = _RPW // _CHUNK
_NSUB = _CHUNK // _GSUB


def _sinusoidal_pe_np(length, d_model):
    pos = np.arange(length, dtype=np.float32)[:, None]
    div = np.exp(
        np.arange(0, d_model, 2, dtype=np.float32) * (-np.log(10000.0) / d_model)
    )
    pe = np.zeros((length, d_model), dtype=np.float32)
    pe[:, 0::2] = np.sin(pos * div)
    pe[:, 1::2] = np.cos(pos * div)
    return pe


_mesh = plsc.VectorSubcoreMesh(core_axis_name="c", subcore_axis_name="s")


@functools.partial(
    pl.kernel,
    mesh=_mesh,
    compiler_params=pltpu.CompilerParams(
        use_tc_tiling_on_sc=False, needs_layout_passes=False),
    out_type=jax.ShapeDtypeStruct((_ROWS // _CHUNK, _L, 2 * _D), jnp.float32),
    scratch_types=[
        pltpu.VMEM((_NCHUNK * _NSUB, _GSUB), jnp.int32),
        pltpu.VMEM((_L, _D), jnp.float32),
        pltpu.VMEM((2, _CHUNK, _D), jnp.float32),
        pltpu.VMEM((2, _CHUNK // 2, 2 * _D), jnp.float32),
        pltpu.SemaphoreType.DMA,
        pltpu.SemaphoreType.DMA,
        pltpu.SemaphoreType.DMA,
        pltpu.SemaphoreType.DMA,
    ],
)
def _embed_kernel(idx_hbm, pe_hbm, table_hbm, out_hbm,
                  idx_v, pe_v, inb, outb, g0, g1, s0, s1):
    wid = lax.axis_index("s") * 2 + lax.axis_index("c")
    cbase = wid * _NCHUNK
    gsem = (g0, g1)
    ssem = (s0, s1)

    pltpu.sync_copy(idx_hbm.at[wid], idx_v)
    pltpu.sync_copy(pe_hbm, pe_v)

    def gather(i, b):
        for s in range(_NSUB):
            pltpu.async_copy(table_hbm.at[idx_v.at[_NSUB * i + s]],
                             inb.at[b, pl.ds(s * _GSUB, _GSUB)], gsem[b])

    def wait_gather(b):
        for _ in range(_NSUB):
            pltpu.make_async_copy(table_hbm.at[idx_v.at[0]],
                                  inb.at[b, pl.ds(0, _GSUB)], gsem[b]).wait()

    def store(i, b):
        pltpu.async_copy(outb.at[b], out_hbm.at[cbase + i], ssem[b])

    def wait_store(b):
        pltpu.make_async_copy(outb.at[b], out_hbm.at[cbase], ssem[b]).wait()

    def add_pe(b):
        def rowpair(off, moff):
            def body(m, _):
                for h in range(2):
                    l = 2 * m + h
                    for q in range(_D // 16):
                        sl = pl.ds(q * 16, 16)
                        osl = pl.ds(h * _D + q * 16, 16)
                        outb[b, moff + m, osl] = inb[b, off + l, sl] + pe_v[l, sl]
                return 0

            lax.fori_loop(0, _L // 2, body, 0)

        rowpair(0, 0)
        rowpair(_L, _L // 2)

    gather(0, 0)
    gather(1, 1)
    for b in (0, 1):
        wait_gather(b)
        add_pe(b)
        store(b, b)
        gather(b + 2, b)

    def body(i2, _):
        for b in (0, 1):
            i = 2 * i2 + b
            wait_gather(b)
            wait_store(b)
            add_pe(b)
            store(i, b)
            gather(i + 2, b)
        return 0

    lax.fori_loop(1, _NCHUNK // 2 - 1, body, 0)

    for b in (0, 1):
        i = _NCHUNK - 2 + b
        wait_gather(b)
        wait_store(b)
        add_pe(b)
        store(i, b)
    wait_store(0)
    wait_store(1)


def kernel(sequence, token_table):
    seq2 = (sequence.reshape(-1).astype(jnp.int32) * 2).reshape(
        _NW, _NCHUNK * _NSUB, _GSUB)
    pe = jnp.asarray(_sinusoidal_pe_np(_L, _D))
    t2m = jnp.pad(token_table, ((0, 0), (0, _D))).reshape(2 * _VOCAB, _D)
    out = _embed_kernel(seq2, pe, t2m)
    return out.reshape(_B, _L, _D)
